# baseline (speedup 1.0000x reference)
import jax
import jax.numpy as jnp
from jax import lax
from jax.experimental import pallas as pl
from jax.experimental.pallas import tpu as pltpu
from jax.experimental.pallas import tpu_sc as plsc

_B, _S, _D = 4, 8192, 768
_NC, _NS = 2, 16
_NW = _NC * _NS
_S_PER_W = _S // _NW
_CS = 64
_NCHUNK = _S_PER_W // _CS
_LANES = 16
_GROUPS = _D // _LANES


def _sc_body(in_hbm, tbl_hbm, out_hbm, buf, tbl):
    wid = lax.axis_index("s") * _NC + lax.axis_index("c")
    s_base = wid * _S_PER_W

    def chunk_body(ci, carry):
        s0 = s_base + ci * _CS
        pltpu.sync_copy(tbl_hbm.at[pl.ds(s0, _CS)], tbl)
        for b in range(_B):
            pltpu.sync_copy(in_hbm.at[b, pl.ds(s0, _CS)], buf)

            def row_body(r, c2):
                for g in range(_GROUPS):
                    plsc.addupdate(
                        buf.at[r, pl.ds(g * _LANES, _LANES)],
                        tbl[r, pl.ds(g * _LANES, _LANES)],
                    )
                return c2

            lax.fori_loop(0, _CS, row_body, 0)
            pltpu.sync_copy(buf, out_hbm.at[b, pl.ds(s0, _CS)])
        return carry

    lax.fori_loop(0, _NCHUNK, chunk_body, 0)


_sc_call = pl.kernel(
    _sc_body,
    out_type=jax.ShapeDtypeStruct((_B, _S, _D), jnp.float32),
    mesh=plsc.VectorSubcoreMesh(core_axis_name="c", subcore_axis_name="s"),
    scratch_types=[
        pltpu.VMEM((_CS, _D), jnp.float32),
        pltpu.VMEM((_CS, _D), jnp.float32),
    ],
)


def kernel(inputs, pos_table):
    return _sc_call(inputs, pos_table)


# speedup vs baseline: 1.5524x; 1.5524x over previous
import jax
import jax.numpy as jnp
from jax import lax
from jax.experimental import pallas as pl
from jax.experimental.pallas import tpu as pltpu
from jax.experimental.pallas import tpu_sc as plsc

_B, _S, _D = 4, 8192, 768
_NC, _NS = 2, 16
_NW = _NC * _NS
_S_PER_W = _S // _NW
_CS = 16
_NCHUNK = _S_PER_W // _CS
_LANES = 16
_GROUPS = _D // _LANES


def _sc_body(in_hbm, tbl_hbm, out_hbm,
             d0, d1, d2, d3, t0, t1,
             ls0, ls1, ls2, ls3, ss0, ss1, ss2, ss3, ts0, ts1):
    dbufs = (d0, d1, d2, d3)
    tbls = (t0, t1)
    lsems = (ls0, ls1, ls2, ls3)
    ssems = (ss0, ss1, ss2, ss3)
    tsems = (ts0, ts1)

    wid = lax.axis_index("s") * _NC + lax.axis_index("c")
    s_base = wid * _S_PER_W

    def chunk_off(c):
        return s_base + lax.rem(c, _NCHUNK) * _CS

    def issue_load(c, b, i):
        pltpu.async_copy(in_hbm.at[b, pl.ds(chunk_off(c), _CS)], dbufs[i],
                         lsems[i])

    def issue_store(c, b, i):
        pltpu.async_copy(dbufs[i], out_hbm.at[b, pl.ds(chunk_off(c), _CS)],
                         ssems[i])

    def issue_tbl(c, j):
        pltpu.async_copy(tbl_hbm.at[pl.ds(chunk_off(c), _CS)], tbls[j],
                         tsems[j])

    def wait_load(i):
        pltpu.make_async_copy(in_hbm.at[0, pl.ds(s_base, _CS)], dbufs[i],
                              lsems[i]).wait()

    def wait_store(i):
        pltpu.make_async_copy(dbufs[i], out_hbm.at[0, pl.ds(s_base, _CS)],
                              ssems[i]).wait()

    def wait_tbl(j):
        pltpu.make_async_copy(tbl_hbm.at[pl.ds(s_base, _CS)], tbls[j],
                              tsems[j]).wait()

    def compute(i, j):
        def row(r, carry):
            for g in range(_GROUPS):
                plsc.addupdate(
                    dbufs[i].at[r, pl.ds(g * _LANES, _LANES)],
                    tbls[j][r, pl.ds(g * _LANES, _LANES)],
                )
            return carry
        lax.fori_loop(0, _CS, row, 0)

    def step(c, k, b, skip_store_wait=False):
        i = b
        j = k
        if b == 0:
            wait_tbl(j)
        wait_load(i)
        compute(i, j)
        issue_store(c, b, i)
        i2 = (b + 2) % 4
        if not skip_store_wait:
            wait_store(i2)
        if b < 2:
            issue_load(c, b + 2, i2)
        else:
            issue_load(c + 1, b - 2, i2)
        if b == 3:
            issue_tbl(c + 2, j)

    issue_tbl(0, 0)
    issue_tbl(1, 1)
    issue_load(0, 0, 0)
    issue_load(0, 1, 1)

    for k in range(2):
        for b in range(_B):
            step(k, k, b, skip_store_wait=(k == 0 and b < 2))

    def pair(cc, carry):
        c = 2 * cc
        for k in range(2):
            for b in range(_B):
                step(c + k, k, b)
        return carry

    lax.fori_loop(1, _NCHUNK // 2, pair, 0)

    wait_load(0)
    wait_load(1)
    wait_tbl(0)
    wait_tbl(1)
    wait_store(2)
    wait_store(3)


_sc_call = pl.kernel(
    _sc_body,
    out_type=jax.ShapeDtypeStruct((_B, _S, _D), jnp.float32),
    mesh=plsc.VectorSubcoreMesh(core_axis_name="c", subcore_axis_name="s"),
    scratch_types=(
        [pltpu.VMEM((_CS, _D), jnp.float32)] * 4
        + [pltpu.VMEM((_CS, _D), jnp.float32)] * 2
        + [pltpu.SemaphoreType.DMA] * 10
    ),
)


def kernel(inputs, pos_table):
    return _sc_call(inputs, pos_table)


# speedup vs baseline: 1.6466x; 1.0607x over previous
import jax
import jax.numpy as jnp
from jax import lax
from jax.experimental import pallas as pl
from jax.experimental.pallas import tpu as pltpu
from jax.experimental.pallas import tpu_sc as plsc

_B, _S, _D = 4, 8192, 768
_NC, _NS = 2, 16
_NW = _NC * _NS
_S_PER_W = _S // _NW
_CS = 32
_NCHUNK = _S_PER_W // _CS
_T = _NCHUNK * _B
_LANES = 16
_GROUPS = _D // _LANES


def _sc_body(in_hbm, tbl_hbm, out_hbm,
             d0, d1, d2, t0, t1,
             ls0, ls1, ls2, ss0, ss1, ss2, ts0, ts1):
    dbufs = (d0, d1, d2)
    tbls = (t0, t1)
    lsems = (ls0, ls1, ls2)
    ssems = (ss0, ss1, ss2)
    tsems = (ts0, ts1)

    wid = lax.axis_index("s") * _NC + lax.axis_index("c")
    s_base = wid * _S_PER_W

    def issue_load(t):
        c, b, i = t // _B, t % _B, t % 3
        pltpu.async_copy(in_hbm.at[b, pl.ds(s_base + c * _CS, _CS)],
                         dbufs[i], lsems[i])

    def issue_store(t):
        c, b, i = t // _B, t % _B, t % 3
        pltpu.async_copy(dbufs[i], out_hbm.at[b, pl.ds(s_base + c * _CS, _CS)],
                         ssems[i])

    def issue_tbl(c):
        j = c % 2
        pltpu.async_copy(tbl_hbm.at[pl.ds(s_base + c * _CS, _CS)],
                         tbls[j], tsems[j])

    def wait_load(i):
        pltpu.make_async_copy(in_hbm.at[0, pl.ds(s_base, _CS)], dbufs[i],
                              lsems[i]).wait()

    def wait_store(i):
        pltpu.make_async_copy(dbufs[i], out_hbm.at[0, pl.ds(s_base, _CS)],
                              ssems[i]).wait()

    def wait_tbl(j):
        pltpu.make_async_copy(tbl_hbm.at[pl.ds(s_base, _CS)], tbls[j],
                              tsems[j]).wait()

    def compute(i, j):
        def row(r, carry):
            for g in range(_GROUPS):
                plsc.addupdate(
                    dbufs[i].at[r, pl.ds(g * _LANES, _LANES)],
                    tbls[j][r, pl.ds(g * _LANES, _LANES)],
                )
            return carry
        lax.fori_loop(0, _CS, row, 0)

    issue_tbl(0)
    issue_tbl(1)
    for t in range(3):
        issue_load(t)

    for t in range(_T):
        c, b, i = t // _B, t % _B, t % 3
        if b == 0:
            wait_tbl(c % 2)
        wait_load(i)
        compute(i, c % 2)
        issue_store(t)
        if b == 3 and c + 2 < _NCHUNK:
            issue_tbl(c + 2)
        if t >= 1:
            wait_store((t - 1) % 3)
            if t + 2 < _T:
                issue_load(t + 2)

    wait_store((_T - 1) % 3)


_sc_call = pl.kernel(
    _sc_body,
    out_type=jax.ShapeDtypeStruct((_B, _S, _D), jnp.float32),
    mesh=plsc.VectorSubcoreMesh(core_axis_name="c", subcore_axis_name="s"),
    scratch_types=(
        [pltpu.VMEM((_CS, _D), jnp.float32)] * 3
        + [pltpu.VMEM((_CS, _D), jnp.float32)] * 2
        + [pltpu.SemaphoreType.DMA] * 8
    ),
)


def kernel(inputs, pos_table):
    return _sc_call(inputs, pos_table)
